# baseline (speedup 1.0000x reference)
import jax
import jax.numpy as jnp
from jax import lax
from jax.experimental import pallas as pl
from jax.experimental.pallas import tpu as pltpu
from jax.experimental.pallas import tpu_sc as plsc

N = 10000
E = 160000
NTILES = 16
EPT = 10240
EPAD = EPT * NTILES
CH = 128
NCH = EPT // CH
F = 128
ACC_ROWS = 10240
ROWS_PT = ACC_ROWS // NTILES
RB = 10000


def _zero_rows(zbuf, nrows):
    zv = jnp.zeros((16,), jnp.float32)

    def zb(r, _):
        for j in range(8):
            zbuf[r, pl.ds(j * 16, 16)] = zv
        return 0

    lax.fori_loop(0, nrows, zb, 0)


G = 40
NG = NCH // G


def _agg_body(x2, src2r, dstr, out, acc, idx_s, idx_d, rb0, rb1, zbuf,
              s0, s1, zsem):
    c = lax.axis_index("c")
    s = lax.axis_index("s")
    _zero_rows(zbuf, 16)

    def zacc(t, _):
        for u in range(4):
            pltpu.async_copy(
                zbuf, acc.at[pl.ds(s * ROWS_PT + (t * 4 + u) * 16, 16)], zsem)
        for u in range(4):
            pltpu.make_async_copy(zbuf, acc.at[pl.ds(0, 16)], zsem).wait()
        return 0

    lax.fori_loop(0, ROWS_PT // 64, zacc, 0)
    plsc.subcore_barrier()

    rbs = (rb0, rb1)
    sems = (s0, s1)

    def group(g, _):
        base = s * NCH + g * G
        pltpu.sync_copy(src2r.at[c, pl.ds(base, G)], idx_s)
        pltpu.sync_copy(dstr.at[pl.ds(base, G)], idx_d)
        for b in range(2):
            pltpu.async_copy(x2.at[idx_s.at[b]], rbs[b], sems[b])

        def inner(k2, _):
            for b in range(2):
                k = k2 * 2 + b
                pltpu.make_async_copy(x2.at[idx_s.at[0]], rbs[b],
                                      sems[b]).wait()
                pltpu.sync_copy(rbs[b], acc.at[idx_d.at[k]], add=True)
                pltpu.async_copy(x2.at[idx_s.at[k + 2]], rbs[b], sems[b])
            return 0

        lax.fori_loop(0, G // 2 - 1, inner, 0)
        for b in range(2):
            pltpu.make_async_copy(x2.at[idx_s.at[0]], rbs[b], sems[b]).wait()
            pltpu.sync_copy(rbs[b], acc.at[idx_d.at[G - 2 + b]], add=True)
        return 0

    lax.fori_loop(0, NG, group, 0)
    plsc.subcore_barrier()
    pltpu.sync_copy(acc.at[pl.ds(s * ROWS_PT, ROWS_PT)],
                    out.at[c, pl.ds(s * ROWS_PT, ROWS_PT)])


_agg_call = pl.kernel(
    _agg_body,
    out_type=jax.ShapeDtypeStruct((2, ACC_ROWS, F), jnp.float32),
    mesh=plsc.VectorSubcoreMesh(core_axis_name="c", subcore_axis_name="s"),
    scratch_types=[
        pltpu.VMEM_SHARED((ACC_ROWS, F), jnp.float32),
        pltpu.VMEM((G, CH), jnp.int32),
        pltpu.VMEM((G, CH), jnp.int32),
        pltpu.VMEM((CH, F), jnp.float32),
        pltpu.VMEM((CH, F), jnp.float32),
        pltpu.VMEM((16, F), jnp.float32),
        pltpu.SemaphoreType.DMA,
        pltpu.SemaphoreType.DMA,
        pltpu.SemaphoreType.DMA,
    ],
)


CNT_RPT = (EPAD // CH) // 32


def _cnt_body(dstr, out, acc, idx_d, onesbuf, zbuf):
    c = lax.axis_index("c")
    s = lax.axis_index("s")
    ov = jnp.full((16,), 1.0, jnp.float32)
    zv = jnp.zeros((16,), jnp.float32)
    for j in range(8):
        onesbuf[pl.ds(j * 16, 16)] = ov

    def zb(t, _):
        zbuf[pl.ds(t * 16, 16)] = zv
        return 0

    lax.fori_loop(0, ROWS_PT // 16, zb, 0)
    pltpu.sync_copy(zbuf, acc.at[pl.ds(s * ROWS_PT, ROWS_PT)])
    base = c * (16 * CNT_RPT) + s * CNT_RPT
    pltpu.sync_copy(dstr.at[pl.ds(base, CNT_RPT)], idx_d)
    plsc.subcore_barrier()

    def ch(k, _):
        pltpu.sync_copy(onesbuf, acc.at[idx_d.at[k]], add=True)
        return 0

    lax.fori_loop(0, CNT_RPT, ch, 0)
    plsc.subcore_barrier()
    pltpu.sync_copy(acc.at[pl.ds(s * ROWS_PT, ROWS_PT)],
                    out.at[c, pl.ds(s * ROWS_PT, ROWS_PT)])


_cnt_call = pl.kernel(
    _cnt_body,
    out_type=jax.ShapeDtypeStruct((2, ACC_ROWS), jnp.float32),
    mesh=plsc.VectorSubcoreMesh(core_axis_name="c", subcore_axis_name="s"),
    scratch_types=[
        pltpu.VMEM_SHARED((ACC_ROWS,), jnp.float32),
        pltpu.VMEM((CNT_RPT, CH), jnp.int32),
        pltpu.VMEM((CH,), jnp.float32),
        pltpu.VMEM((ROWS_PT,), jnp.float32),
    ],
)


def _enc_body(xf_ref, xop_ref, emb_ref, w0a_ref, w0b_ref, b0_ref, out_ref):
    oh = (xop_ref[...] == lax.broadcasted_iota(jnp.int32, (RB, 120), 1)
          ).astype(jnp.float32)
    e_tab = emb_ref[...] @ w0b_ref[...]
    res = xf_ref[...] @ w0a_ref[...] + oh @ e_tab + b0_ref[...]
    out_ref[0] = res[:, :F]
    out_ref[1] = res[:, F:]


def _encoder(x_feat, xop2d, emb_pad, w0a, w0b_pad, b0r):
    return pl.pallas_call(
        _enc_body,
        grid=(N // RB,),
        in_specs=[
            pl.BlockSpec((RB, 140), lambda i: (i, 0)),
            pl.BlockSpec((RB, 1), lambda i: (i, 0)),
            pl.BlockSpec((120, 8), lambda i: (0, 0)),
            pl.BlockSpec((140, 256), lambda i: (0, 0)),
            pl.BlockSpec((8, 256), lambda i: (0, 0)),
            pl.BlockSpec((1, 256), lambda i: (0, 0)),
        ],
        out_specs=pl.BlockSpec((2, RB, F), lambda i: (0, i, 0)),
        out_shape=jax.ShapeDtypeStruct((2, N, F), jnp.float32),
    )(x_feat, xop2d, emb_pad, w0a, w0b_pad, b0r)


def _rpart_body(x_ref, wr_ref, bl_ref, out_ref):
    r = (x_ref[0] @ wr_ref[pl.ds(0, F)] + x_ref[1] @ wr_ref[pl.ds(F, F)]
         + bl_ref[...])
    out_ref[0] = r[:, :F]
    out_ref[1] = r[:, F:]


def _rpart(x, wr, blr):
    return pl.pallas_call(
        _rpart_body,
        grid=(N // RB,),
        in_specs=[
            pl.BlockSpec((2, RB, F), lambda i: (0, i, 0)),
            pl.BlockSpec((256, 256), lambda i: (0, 0)),
            pl.BlockSpec((1, 256), lambda i: (0, 0)),
        ],
        out_specs=pl.BlockSpec((2, RB, F), lambda i: (0, i, 0)),
        out_shape=jax.ShapeDtypeStruct((2, N, F), jnp.float32),
    )(x, wr, blr)


def _combine_body(agg_ref, cnt_ref, r_ref, wl_ref, out_ref):
    inv = 1.0 / jnp.maximum(cnt_ref[0] + cnt_ref[1], 1.0)
    h = ((agg_ref[0] * inv) @ wl_ref[pl.ds(0, F)]
         + (agg_ref[1] * inv) @ wl_ref[pl.ds(F, F)])
    y0 = jnp.maximum(h[:, :F] + r_ref[0], 0.0)
    y1 = jnp.maximum(h[:, F:] + r_ref[1], 0.0)
    out_ref[0] = y0
    out_ref[1] = y1


def _combine(agg, cnt3, r, wl):
    return pl.pallas_call(
        _combine_body,
        grid=(N // RB,),
        in_specs=[
            pl.BlockSpec((2, RB, F), lambda i: (0, i, 0)),
            pl.BlockSpec((2, RB, 1), lambda i: (0, i, 0)),
            pl.BlockSpec((2, RB, F), lambda i: (0, i, 0)),
            pl.BlockSpec((256, 256), lambda i: (0, 0)),
        ],
        out_specs=pl.BlockSpec((2, RB, F), lambda i: (0, i, 0)),
        out_shape=jax.ShapeDtypeStruct((2, N, F), jnp.float32),
    )(agg, cnt3, r, wl)


def _head_body(x_ref, xcfg_ref, d1a_ref, d1b_ref, d1c_ref, bd1_ref,
               d2_ref, bd2_ref, d3_ref, bd3_ref, out_ref, ssum, smax):
    i = pl.program_id(0)
    xb = jnp.concatenate([x_ref[0], x_ref[1]], axis=1)
    ps = jnp.sum(xb, axis=0, keepdims=True)
    pm = jnp.max(xb, axis=0, keepdims=True)

    @pl.when(i == 0)
    def _():
        ssum[...] = ps
        smax[...] = pm

    @pl.when(i > 0)
    def _():
        ssum[...] = ssum[...] + ps
        smax[...] = jnp.maximum(smax[...], pm)

    @pl.when(i == (N // RB) - 1)
    def _():
        xmean = ssum[...] * (1.0 / N)
        xmax = smax[...]
        h1 = jnp.maximum(
            xcfg_ref[...] @ d1a_ref[...] + xmax @ d1b_ref[...]
            + xmean @ d1c_ref[...] + bd1_ref[...], 0.0)
        h2 = jnp.maximum(h1 @ d2_ref[...] + bd2_ref[...], 0.0)
        o = h2 @ d3_ref[...] + bd3_ref[...]
        mu = jnp.mean(o)
        sd = jnp.sqrt(jnp.sum((o - mu) ** 2) * (1.0 / 999.0))
        out_ref[...] = (o - mu) / (sd + 1e-5)


def _head(x, x_cfg, d1a, d1b, d1c, bd1r, d2, bd2r, d3, bd3r):
    return pl.pallas_call(
        _head_body,
        grid=(N // RB,),
        in_specs=[
            pl.BlockSpec((2, RB, F), lambda i: (0, i, 0)),
            pl.BlockSpec((1000, 24), lambda i: (0, 0)),
            pl.BlockSpec((24, 512), lambda i: (0, 0)),
            pl.BlockSpec((256, 512), lambda i: (0, 0)),
            pl.BlockSpec((256, 512), lambda i: (0, 0)),
            pl.BlockSpec((1, 512), lambda i: (0, 0)),
            pl.BlockSpec((512, 512), lambda i: (0, 0)),
            pl.BlockSpec((1, 512), lambda i: (0, 0)),
            pl.BlockSpec((512, 1), lambda i: (0, 0)),
            pl.BlockSpec((1, 1), lambda i: (0, 0)),
        ],
        out_specs=pl.BlockSpec((1000, 1), lambda i: (0, 0)),
        out_shape=jax.ShapeDtypeStruct((1000, 1), jnp.float32),
        scratch_shapes=[
            pltpu.VMEM((1, 256), jnp.float32),
            pltpu.VMEM((1, 256), jnp.float32),
        ],
    )(x, x_cfg, d1a, d1b, d1c, bd1r, d2, bd2r, d3, bd3r)


def kernel(x_cfg, x_feat, x_op, edge_index, emb, W0, b0, Wl0, bl0, Wr0,
           Wl1, bl1, Wr1, Wl2, bl2, Wr2, D1, bd1, D2, bd2, D3, bd3):
    src = edge_index[0].astype(jnp.int32)
    dst = edge_index[1].astype(jnp.int32)
    pe = EPAD - E
    srcp = jnp.concatenate([src, jnp.zeros((pe,), jnp.int32)])
    src2r = jnp.stack([srcp, srcp + N]).reshape(2, EPAD // CH, CH)
    dstp = jnp.concatenate([dst, jnp.full((pe,), N, jnp.int32)])
    dstr = dstp.reshape(EPAD // CH, CH)

    cntp = _cnt_call(dstr)
    cnt3 = cntp.reshape(2, ACC_ROWS, 1)

    xop2d = x_op.astype(jnp.int32).reshape(N, 1)
    emb_pad = jnp.concatenate(
        [emb, jnp.zeros((120, 4), jnp.float32)], axis=1)
    w0a = W0[:140]
    w0b_pad = jnp.concatenate(
        [W0[140:], jnp.zeros((4, 256), jnp.float32)], axis=0)
    x = _encoder(x_feat, xop2d, emb_pad, w0a, w0b_pad, b0.reshape(1, 256))

    for wl, bl, wr in ((Wl0, bl0, Wr0), (Wl1, bl1, Wr1), (Wl2, bl2, Wr2)):
        agg = _agg_call(x.reshape(2 * N, F), src2r, dstr)
        r = _rpart(x, wr, bl.reshape(1, 256))
        x = _combine(agg, cnt3, r, wl)

    out = _head(x, x_cfg, D1[:24], D1[24:280], D1[280:536],
                bd1.reshape(1, 512), D2, bd2.reshape(1, 512),
                D3, bd3.reshape(1, 1))
    return out.reshape(-1)


# speedup vs baseline: 1.0093x; 1.0093x over previous
import jax
import jax.numpy as jnp
from jax import lax
from jax.experimental import pallas as pl
from jax.experimental.pallas import tpu as pltpu
from jax.experimental.pallas import tpu_sc as plsc

N = 10000
E = 160000
NTILES = 16
EPT = 10240
EPAD = EPT * NTILES
CH = 128
NCH = EPT // CH
F = 128
ACC_ROWS = 10240
ROWS_PT = ACC_ROWS // NTILES
RB = 5000


def _zero_rows(zbuf, nrows):
    zv = jnp.zeros((16,), jnp.float32)

    def zb(r, _):
        for j in range(8):
            zbuf[r, pl.ds(j * 16, 16)] = zv
        return 0

    lax.fori_loop(0, nrows, zb, 0)


G = 40
NG = NCH // G


def _agg_body(x2, src2r, dstr, out, acc, idx_s, idx_d, rb0, rb1, zbuf,
              s0, s1, zsem):
    c = lax.axis_index("c")
    s = lax.axis_index("s")
    _zero_rows(zbuf, 16)

    def zacc(t, _):
        for u in range(4):
            pltpu.async_copy(
                zbuf, acc.at[pl.ds(s * ROWS_PT + (t * 4 + u) * 16, 16)], zsem)
        for u in range(4):
            pltpu.make_async_copy(zbuf, acc.at[pl.ds(0, 16)], zsem).wait()
        return 0

    lax.fori_loop(0, ROWS_PT // 64, zacc, 0)
    plsc.subcore_barrier()

    rbs = (rb0, rb1)
    sems = (s0, s1)

    def group(g, _):
        base = s * NCH + g * G
        pltpu.sync_copy(src2r.at[c, pl.ds(base, G)], idx_s)
        pltpu.sync_copy(dstr.at[pl.ds(base, G)], idx_d)
        for b in range(2):
            pltpu.async_copy(x2.at[idx_s.at[b]], rbs[b], sems[b])

        def inner(k2, _):
            for b in range(2):
                k = k2 * 2 + b
                pltpu.make_async_copy(x2.at[idx_s.at[0]], rbs[b],
                                      sems[b]).wait()
                pltpu.sync_copy(rbs[b], acc.at[idx_d.at[k]], add=True)
                pltpu.async_copy(x2.at[idx_s.at[k + 2]], rbs[b], sems[b])
            return 0

        lax.fori_loop(0, G // 2 - 1, inner, 0)
        for b in range(2):
            pltpu.make_async_copy(x2.at[idx_s.at[0]], rbs[b], sems[b]).wait()
            pltpu.sync_copy(rbs[b], acc.at[idx_d.at[G - 2 + b]], add=True)
        return 0

    lax.fori_loop(0, NG, group, 0)
    plsc.subcore_barrier()
    pltpu.sync_copy(acc.at[pl.ds(s * ROWS_PT, ROWS_PT)],
                    out.at[c, pl.ds(s * ROWS_PT, ROWS_PT)])


_agg_call = pl.kernel(
    _agg_body,
    out_type=jax.ShapeDtypeStruct((2, ACC_ROWS, F), jnp.float32),
    mesh=plsc.VectorSubcoreMesh(core_axis_name="c", subcore_axis_name="s"),
    scratch_types=[
        pltpu.VMEM_SHARED((ACC_ROWS, F), jnp.float32),
        pltpu.VMEM((G, CH), jnp.int32),
        pltpu.VMEM((G, CH), jnp.int32),
        pltpu.VMEM((CH, F), jnp.float32),
        pltpu.VMEM((CH, F), jnp.float32),
        pltpu.VMEM((16, F), jnp.float32),
        pltpu.SemaphoreType.DMA,
        pltpu.SemaphoreType.DMA,
        pltpu.SemaphoreType.DMA,
    ],
)


CNT_RPT = (EPAD // CH) // 32


def _cnt_body(dstr, out, acc, idx_d, onesbuf, zbuf):
    c = lax.axis_index("c")
    s = lax.axis_index("s")
    ov = jnp.full((16,), 1.0, jnp.float32)
    zv = jnp.zeros((16,), jnp.float32)
    for j in range(8):
        onesbuf[pl.ds(j * 16, 16)] = ov

    def zb(t, _):
        zbuf[pl.ds(t * 16, 16)] = zv
        return 0

    lax.fori_loop(0, ROWS_PT // 16, zb, 0)
    pltpu.sync_copy(zbuf, acc.at[pl.ds(s * ROWS_PT, ROWS_PT)])
    base = c * (16 * CNT_RPT) + s * CNT_RPT
    pltpu.sync_copy(dstr.at[pl.ds(base, CNT_RPT)], idx_d)
    plsc.subcore_barrier()

    def ch(k, _):
        pltpu.sync_copy(onesbuf, acc.at[idx_d.at[k]], add=True)
        return 0

    lax.fori_loop(0, CNT_RPT, ch, 0)
    plsc.subcore_barrier()
    pltpu.sync_copy(acc.at[pl.ds(s * ROWS_PT, ROWS_PT)],
                    out.at[c, pl.ds(s * ROWS_PT, ROWS_PT)])


_cnt_call = pl.kernel(
    _cnt_body,
    out_type=jax.ShapeDtypeStruct((2, ACC_ROWS), jnp.float32),
    mesh=plsc.VectorSubcoreMesh(core_axis_name="c", subcore_axis_name="s"),
    scratch_types=[
        pltpu.VMEM_SHARED((ACC_ROWS,), jnp.float32),
        pltpu.VMEM((CNT_RPT, CH), jnp.int32),
        pltpu.VMEM((CH,), jnp.float32),
        pltpu.VMEM((ROWS_PT,), jnp.float32),
    ],
)


def _enc_body(xf_ref, xop_ref, emb_ref, w0a_ref, w0b_ref, b0_ref, out_ref):
    oh = (xop_ref[...] == lax.broadcasted_iota(jnp.int32, (RB, 120), 1)
          ).astype(jnp.float32)
    e_tab = emb_ref[...] @ w0b_ref[...]
    res = xf_ref[...] @ w0a_ref[...] + oh @ e_tab + b0_ref[...]
    out_ref[0] = res[:, :F]
    out_ref[1] = res[:, F:]


def _encoder(x_feat, xop2d, emb_pad, w0a, w0b_pad, b0r):
    return pl.pallas_call(
        _enc_body,
        grid=(N // RB,),
        in_specs=[
            pl.BlockSpec((RB, 140), lambda i: (i, 0)),
            pl.BlockSpec((RB, 1), lambda i: (i, 0)),
            pl.BlockSpec((120, 8), lambda i: (0, 0)),
            pl.BlockSpec((140, 256), lambda i: (0, 0)),
            pl.BlockSpec((8, 256), lambda i: (0, 0)),
            pl.BlockSpec((1, 256), lambda i: (0, 0)),
        ],
        out_specs=pl.BlockSpec((2, RB, F), lambda i: (0, i, 0)),
        out_shape=jax.ShapeDtypeStruct((2, N, F), jnp.float32),
    )(x_feat, xop2d, emb_pad, w0a, w0b_pad, b0r)


def _rpart_body(x_ref, wr_ref, bl_ref, out_ref):
    r = (x_ref[0] @ wr_ref[pl.ds(0, F)] + x_ref[1] @ wr_ref[pl.ds(F, F)]
         + bl_ref[...])
    out_ref[0] = r[:, :F]
    out_ref[1] = r[:, F:]


def _rpart(x, wr, blr):
    return pl.pallas_call(
        _rpart_body,
        grid=(N // RB,),
        in_specs=[
            pl.BlockSpec((2, RB, F), lambda i: (0, i, 0)),
            pl.BlockSpec((256, 256), lambda i: (0, 0)),
            pl.BlockSpec((1, 256), lambda i: (0, 0)),
        ],
        out_specs=pl.BlockSpec((2, RB, F), lambda i: (0, i, 0)),
        out_shape=jax.ShapeDtypeStruct((2, N, F), jnp.float32),
    )(x, wr, blr)


def _combine_body(agg_ref, cnt_ref, r_ref, wl_ref, out_ref):
    inv = 1.0 / jnp.maximum(cnt_ref[0] + cnt_ref[1], 1.0)
    h = ((agg_ref[0] * inv) @ wl_ref[pl.ds(0, F)]
         + (agg_ref[1] * inv) @ wl_ref[pl.ds(F, F)])
    y0 = jnp.maximum(h[:, :F] + r_ref[0], 0.0)
    y1 = jnp.maximum(h[:, F:] + r_ref[1], 0.0)
    out_ref[0] = y0
    out_ref[1] = y1


def _combine(agg, cnt3, r, wl):
    return pl.pallas_call(
        _combine_body,
        grid=(N // RB,),
        in_specs=[
            pl.BlockSpec((2, RB, F), lambda i: (0, i, 0)),
            pl.BlockSpec((2, RB, 1), lambda i: (0, i, 0)),
            pl.BlockSpec((2, RB, F), lambda i: (0, i, 0)),
            pl.BlockSpec((256, 256), lambda i: (0, 0)),
        ],
        out_specs=pl.BlockSpec((2, RB, F), lambda i: (0, i, 0)),
        out_shape=jax.ShapeDtypeStruct((2, N, F), jnp.float32),
    )(agg, cnt3, r, wl)


def _head_body(x_ref, xcfg_ref, d1a_ref, d1b_ref, d1c_ref, bd1_ref,
               d2_ref, bd2_ref, d3_ref, bd3_ref, out_ref, ssum, smax):
    i = pl.program_id(0)
    xb = jnp.concatenate([x_ref[0], x_ref[1]], axis=1)
    ps = jnp.sum(xb, axis=0, keepdims=True)
    pm = jnp.max(xb, axis=0, keepdims=True)

    @pl.when(i == 0)
    def _():
        ssum[...] = ps
        smax[...] = pm

    @pl.when(i > 0)
    def _():
        ssum[...] = ssum[...] + ps
        smax[...] = jnp.maximum(smax[...], pm)

    @pl.when(i == (N // RB) - 1)
    def _():
        xmean = ssum[...] * (1.0 / N)
        xmax = smax[...]
        h1 = jnp.maximum(
            xcfg_ref[...] @ d1a_ref[...] + xmax @ d1b_ref[...]
            + xmean @ d1c_ref[...] + bd1_ref[...], 0.0)
        h2 = jnp.maximum(h1 @ d2_ref[...] + bd2_ref[...], 0.0)
        o = h2 @ d3_ref[...] + bd3_ref[...]
        mu = jnp.mean(o)
        sd = jnp.sqrt(jnp.sum((o - mu) ** 2) * (1.0 / 999.0))
        out_ref[...] = (o - mu) / (sd + 1e-5)


def _head(x, x_cfg, d1a, d1b, d1c, bd1r, d2, bd2r, d3, bd3r):
    return pl.pallas_call(
        _head_body,
        grid=(N // RB,),
        in_specs=[
            pl.BlockSpec((2, RB, F), lambda i: (0, i, 0)),
            pl.BlockSpec((1000, 24), lambda i: (0, 0)),
            pl.BlockSpec((24, 512), lambda i: (0, 0)),
            pl.BlockSpec((256, 512), lambda i: (0, 0)),
            pl.BlockSpec((256, 512), lambda i: (0, 0)),
            pl.BlockSpec((1, 512), lambda i: (0, 0)),
            pl.BlockSpec((512, 512), lambda i: (0, 0)),
            pl.BlockSpec((1, 512), lambda i: (0, 0)),
            pl.BlockSpec((512, 1), lambda i: (0, 0)),
            pl.BlockSpec((1, 1), lambda i: (0, 0)),
        ],
        out_specs=pl.BlockSpec((1000, 1), lambda i: (0, 0)),
        out_shape=jax.ShapeDtypeStruct((1000, 1), jnp.float32),
        scratch_shapes=[
            pltpu.VMEM((1, 256), jnp.float32),
            pltpu.VMEM((1, 256), jnp.float32),
        ],
    )(x, x_cfg, d1a, d1b, d1c, bd1r, d2, bd2r, d3, bd3r)


def kernel(x_cfg, x_feat, x_op, edge_index, emb, W0, b0, Wl0, bl0, Wr0,
           Wl1, bl1, Wr1, Wl2, bl2, Wr2, D1, bd1, D2, bd2, D3, bd3):
    src = edge_index[0].astype(jnp.int32)
    dst = edge_index[1].astype(jnp.int32)
    pe = EPAD - E
    srcp = jnp.concatenate([src, jnp.zeros((pe,), jnp.int32)])
    src2r = jnp.stack([srcp, srcp + N]).reshape(2, EPAD // CH, CH)
    dstp = jnp.concatenate([dst, jnp.full((pe,), N, jnp.int32)])
    dstr = dstp.reshape(EPAD // CH, CH)

    cntp = _cnt_call(dstr)
    cnt3 = cntp.reshape(2, ACC_ROWS, 1)

    xop2d = x_op.astype(jnp.int32).reshape(N, 1)
    emb_pad = jnp.concatenate(
        [emb, jnp.zeros((120, 4), jnp.float32)], axis=1)
    w0a = W0[:140]
    w0b_pad = jnp.concatenate(
        [W0[140:], jnp.zeros((4, 256), jnp.float32)], axis=0)
    x = _encoder(x_feat, xop2d, emb_pad, w0a, w0b_pad, b0.reshape(1, 256))

    for wl, bl, wr in ((Wl0, bl0, Wr0), (Wl1, bl1, Wr1), (Wl2, bl2, Wr2)):
        agg = _agg_call(x.reshape(2 * N, F), src2r, dstr)
        r = _rpart(x, wr, bl.reshape(1, 256))
        x = _combine(agg, cnt3, r, wl)

    out = _head(x, x_cfg, D1[:24], D1[24:280], D1[280:536],
                bd1.reshape(1, 512), D2, bd2.reshape(1, 512),
                D3, bd3.reshape(1, 1))
    return out.reshape(-1)
